# baseline (device time: 567714 ns/iter reference)
import jax
import jax.numpy as jnp
from jax import lax
from jax.experimental import pallas as pl
from jax.experimental.pallas import tpu as pltpu

N_Y = 2
N_CHUNKS = 16
N_ASM = 4


def kernel(x):
    m, n = x.shape
    half_n = n // 2
    half_m = m // 2
    rows_c = half_m // N_CHUNKS

    def comm_body(
        x_hbm, exch_hbm, own_hbm,
        peer_f32, own_f32, own_bf16, send_buf,
        cp_sems, peer_sems, out_sems, ysend, yrecv, xsend, xrecv,
    ):
        my_x = lax.axis_index("x")
        my_y = lax.axis_index("y")
        y_nbr = (my_x, 1 - my_y)
        x_nbr = (1 - my_x, my_y)

        cp_peers = []
        for c in range(N_CHUNKS):
            cp = pltpu.make_async_copy(
                x_hbm.at[pl.ds(my_x * half_m + c * rows_c, rows_c),
                         pl.ds((1 - my_y) * half_n, half_n)],
                peer_f32.at[pl.ds(c * rows_c, rows_c), :],
                peer_sems.at[c],
            )
            cp.start()
            cp_peers.append(cp)

        barrier_sem = pltpu.get_barrier_semaphore()
        for nbr in (y_nbr, x_nbr):
            pl.semaphore_signal(
                barrier_sem, inc=1,
                device_id=nbr, device_id_type=pl.DeviceIdType.MESH,
            )
        pl.semaphore_wait(barrier_sem, 2)

        y_rdmas = []
        for c in range(N_CHUNKS):
            cp_peers[c].wait()
            send_buf[pl.ds(c * rows_c, rows_c), :] = peer_f32[
                pl.ds(c * rows_c, rows_c), :
            ].astype(jnp.bfloat16)
            rd = pltpu.make_async_remote_copy(
                src_ref=send_buf.at[pl.ds(c * rows_c, rows_c), :],
                dst_ref=exch_hbm.at[
                    pl.ds(my_x * half_m + c * rows_c, rows_c), :
                ],
                send_sem=ysend.at[c],
                recv_sem=yrecv.at[c],
                device_id=y_nbr,
                device_id_type=pl.DeviceIdType.MESH,
            )
            rd.start()
            y_rdmas.append(rd)

        cp_own = pltpu.make_async_copy(
            x_hbm.at[:, pl.ds(my_y * half_n, half_n)],
            own_f32,
            cp_sems.at[0],
        )
        cp_own.start()

        own_rows = m // N_CHUNKS
        x_rdmas = []
        stores = []
        for c in range(N_CHUNKS):
            row0 = my_x * half_m + c * rows_c
            recv_view = pltpu.make_async_remote_copy(
                src_ref=send_buf.at[pl.ds(c * rows_c, rows_c), :],
                dst_ref=exch_hbm.at[pl.ds(row0, rows_c), :],
                send_sem=ysend.at[c],
                recv_sem=yrecv.at[c],
                device_id=y_nbr,
                device_id_type=pl.DeviceIdType.MESH,
            )
            recv_view.wait_recv()
            fwd = pltpu.make_async_remote_copy(
                src_ref=exch_hbm.at[pl.ds(row0, rows_c), :],
                dst_ref=exch_hbm.at[pl.ds(row0, rows_c), :],
                send_sem=xsend.at[c],
                recv_sem=xrecv.at[c],
                device_id=x_nbr,
                device_id_type=pl.DeviceIdType.MESH,
            )
            fwd.start()
            x_rdmas.append(fwd)

            if c == 0:
                cp_own.wait()
            own_bf16[pl.ds(c * own_rows, own_rows), :] = own_f32[
                pl.ds(c * own_rows, own_rows), :
            ].astype(jnp.bfloat16)
            st = pltpu.make_async_copy(
                own_bf16.at[pl.ds(c * own_rows, own_rows), :],
                own_hbm.at[pl.ds(c * own_rows, own_rows), :],
                out_sems.at[c],
            )
            st.start()
            stores.append(st)

        for c in range(N_CHUNKS):
            x_rdmas[c].wait_recv()
        for c in range(N_CHUNKS):
            y_rdmas[c].wait_send()
            x_rdmas[c].wait_send()
            stores[c].wait()

    exch, own = pl.pallas_call(
        comm_body,
        out_shape=[
            jax.ShapeDtypeStruct((m, half_n), jnp.bfloat16),
            jax.ShapeDtypeStruct((m, half_n), jnp.bfloat16),
        ],
        in_specs=[pl.BlockSpec(memory_space=pl.ANY)],
        out_specs=[pl.BlockSpec(memory_space=pl.ANY),
                   pl.BlockSpec(memory_space=pl.ANY)],
        scratch_shapes=[
            pltpu.VMEM((half_m, half_n), jnp.float32),
            pltpu.VMEM((m, half_n), jnp.float32),
            pltpu.VMEM((m, half_n), jnp.bfloat16),
            pltpu.VMEM((half_m, half_n), jnp.bfloat16),
            pltpu.SemaphoreType.DMA((1,)),
            pltpu.SemaphoreType.DMA((N_CHUNKS,)),
            pltpu.SemaphoreType.DMA((N_CHUNKS,)),
            pltpu.SemaphoreType.DMA((N_CHUNKS,)),
            pltpu.SemaphoreType.DMA((N_CHUNKS,)),
            pltpu.SemaphoreType.DMA((N_CHUNKS,)),
            pltpu.SemaphoreType.DMA((N_CHUNKS,)),
        ],
        compiler_params=pltpu.CompilerParams(
            collective_id=0,
            vmem_limit_bytes=96 * 1024 * 1024,
        ),
    )(x)

    rows_a = m // N_ASM

    def asm_body(exch_hbm, own_hbm, o_hbm, sems):
        my_y = lax.axis_index("y")
        cps = []
        for a in range(N_ASM):
            cp = pltpu.make_async_copy(
                own_hbm.at[pl.ds(a * rows_a, rows_a), :],
                o_hbm.at[pl.ds(my_y * m + a * rows_a, rows_a), :],
                sems.at[a],
            )
            cp.start()
            cps.append(cp)
            cp = pltpu.make_async_copy(
                exch_hbm.at[pl.ds(a * rows_a, rows_a), :],
                o_hbm.at[pl.ds((1 - my_y) * m + a * rows_a, rows_a), :],
                sems.at[N_ASM + a],
            )
            cp.start()
            cps.append(cp)
        for cp in cps:
            cp.wait()

    return pl.pallas_call(
        asm_body,
        out_shape=jax.ShapeDtypeStruct((N_Y * m, half_n), jnp.bfloat16),
        in_specs=[pl.BlockSpec(memory_space=pl.ANY),
                  pl.BlockSpec(memory_space=pl.ANY)],
        out_specs=pl.BlockSpec(memory_space=pl.ANY),
        scratch_shapes=[pltpu.SemaphoreType.DMA((2 * N_ASM,))],
        compiler_params=pltpu.CompilerParams(
            vmem_limit_bytes=96 * 1024 * 1024,
        ),
    )(exch, own)


# device time: 68289 ns/iter; 8.3134x vs baseline; 8.3134x over previous
import jax
import jax.numpy as jnp
from jax import lax
from jax.experimental import pallas as pl
from jax.experimental.pallas import tpu as pltpu

N_Y = 2
N_CHUNKS = 16
N_ASM = 4


def kernel(x):
    m, n = x.shape
    half_n = n // 2
    half_m = m // 2
    rows_c = half_m // N_CHUNKS

    def comm_body(
        x_hbm, exch_hbm, own_hbm,
        peer_f32, own_f32, own_bf16, send_buf,
        cp_sems, peer_sems, out_sems, ysend, yrecv, xsend, xrecv,
    ):
        my_x = lax.axis_index("x")
        my_y = lax.axis_index("y")
        y_nbr = (my_x, 1 - my_y)
        x_nbr = (1 - my_x, my_y)

        cp_peers = []
        for c in range(N_CHUNKS):
            cp = pltpu.make_async_copy(
                x_hbm.at[pl.ds(my_x * half_m + c * rows_c, rows_c),
                         pl.ds((1 - my_y) * half_n, half_n)],
                peer_f32.at[pl.ds(c * rows_c, rows_c), :],
                peer_sems.at[c],
            )
            cp.start()
            cp_peers.append(cp)

        barrier_sem = pltpu.get_barrier_semaphore()
        for nbr in (y_nbr, x_nbr):
            pl.semaphore_signal(
                barrier_sem, inc=1,
                device_id=nbr, device_id_type=pl.DeviceIdType.MESH,
            )
        pl.semaphore_wait(barrier_sem, 2)

        y_rdmas = []
        for c in range(N_CHUNKS):
            cp_peers[c].wait()
            send_buf[pl.ds(c * rows_c, rows_c), :] = peer_f32[
                pl.ds(c * rows_c, rows_c), :
            ].astype(jnp.bfloat16)
            rd = pltpu.make_async_remote_copy(
                src_ref=send_buf.at[pl.ds(c * rows_c, rows_c), :],
                dst_ref=exch_hbm.at[
                    pl.ds(my_x * half_m + c * rows_c, rows_c), :
                ],
                send_sem=ysend.at[c],
                recv_sem=yrecv.at[c],
                device_id=y_nbr,
                device_id_type=pl.DeviceIdType.MESH,
            )
            rd.start()
            y_rdmas.append(rd)

        cp_own = pltpu.make_async_copy(
            x_hbm.at[:, pl.ds(my_y * half_n, half_n)],
            own_f32,
            cp_sems.at[0],
        )
        cp_own.start()

        own_rows = m // N_CHUNKS
        x_rdmas = []
        stores = []
        for c in range(N_CHUNKS):
            row0 = my_x * half_m + c * rows_c
            recv_view = pltpu.make_async_remote_copy(
                src_ref=send_buf.at[pl.ds(c * rows_c, rows_c), :],
                dst_ref=exch_hbm.at[pl.ds(row0, rows_c), :],
                send_sem=ysend.at[c],
                recv_sem=yrecv.at[c],
                device_id=y_nbr,
                device_id_type=pl.DeviceIdType.MESH,
            )
            recv_view.wait_recv()
            fwd = pltpu.make_async_remote_copy(
                src_ref=exch_hbm.at[pl.ds(row0, rows_c), :],
                dst_ref=exch_hbm.at[pl.ds(row0, rows_c), :],
                send_sem=xsend.at[c],
                recv_sem=xrecv.at[c],
                device_id=x_nbr,
                device_id_type=pl.DeviceIdType.MESH,
            )
            fwd.start()
            x_rdmas.append(fwd)

            if c == 0:
                cp_own.wait()
            own_bf16[pl.ds(c * own_rows, own_rows), :] = own_f32[
                pl.ds(c * own_rows, own_rows), :
            ].astype(jnp.bfloat16)
            st = pltpu.make_async_copy(
                own_bf16.at[pl.ds(c * own_rows, own_rows), :],
                own_hbm.at[pl.ds(c * own_rows, own_rows), :],
                out_sems.at[c],
            )
            st.start()
            stores.append(st)

        for c in range(N_CHUNKS):
            x_rdmas[c].wait_recv()
        for c in range(N_CHUNKS):
            y_rdmas[c].wait_send()
            x_rdmas[c].wait_send()
            stores[c].wait()

    exch, own = pl.pallas_call(
        comm_body,
        out_shape=[
            jax.ShapeDtypeStruct((m, half_n), jnp.bfloat16),
            jax.ShapeDtypeStruct((m, half_n), jnp.bfloat16),
        ],
        in_specs=[pl.BlockSpec(memory_space=pl.ANY)],
        out_specs=[pl.BlockSpec(memory_space=pl.ANY),
                   pl.BlockSpec(memory_space=pl.ANY)],
        scratch_shapes=[
            pltpu.VMEM((half_m, half_n), jnp.float32),
            pltpu.VMEM((m, half_n), jnp.float32),
            pltpu.VMEM((m, half_n), jnp.bfloat16),
            pltpu.VMEM((half_m, half_n), jnp.bfloat16),
            pltpu.SemaphoreType.DMA((1,)),
            pltpu.SemaphoreType.DMA((N_CHUNKS,)),
            pltpu.SemaphoreType.DMA((N_CHUNKS,)),
            pltpu.SemaphoreType.DMA((N_CHUNKS,)),
            pltpu.SemaphoreType.DMA((N_CHUNKS,)),
            pltpu.SemaphoreType.DMA((N_CHUNKS,)),
            pltpu.SemaphoreType.DMA((N_CHUNKS,)),
        ],
        compiler_params=pltpu.CompilerParams(
            collective_id=0,
            vmem_limit_bytes=96 * 1024 * 1024,
        ),
    )(x)

    rows_a = m // N_ASM
    n_tasks = 2 * N_ASM

    def asm_body(exch_hbm, own_hbm, o_hbm, bounce, isems, osems):
        my_y = lax.axis_index("y")
        tasks = []
        for a in range(N_ASM):
            tasks.append((own_hbm, a * rows_a, my_y * m + a * rows_a))
        for a in range(N_ASM):
            tasks.append((exch_hbm, a * rows_a, (1 - my_y) * m + a * rows_a))

        ins = []
        for i, (src, s0, _) in enumerate(tasks):
            cp = pltpu.make_async_copy(
                src.at[pl.ds(s0, rows_a), :],
                bounce.at[i],
                isems.at[i],
            )
            cp.start()
            ins.append(cp)
        outs = []
        for i, (_, _, d0) in enumerate(tasks):
            ins[i].wait()
            cp = pltpu.make_async_copy(
                bounce.at[i],
                o_hbm.at[pl.ds(d0, rows_a), :],
                osems.at[i],
            )
            cp.start()
            outs.append(cp)
        for cp in outs:
            cp.wait()

    return pl.pallas_call(
        asm_body,
        out_shape=jax.ShapeDtypeStruct((N_Y * m, half_n), jnp.bfloat16),
        in_specs=[pl.BlockSpec(memory_space=pl.ANY),
                  pl.BlockSpec(memory_space=pl.ANY)],
        out_specs=pl.BlockSpec(memory_space=pl.ANY),
        scratch_shapes=[
            pltpu.VMEM((n_tasks, rows_a, half_n), jnp.bfloat16),
            pltpu.SemaphoreType.DMA((n_tasks,)),
            pltpu.SemaphoreType.DMA((n_tasks,)),
        ],
        compiler_params=pltpu.CompilerParams(
            vmem_limit_bytes=96 * 1024 * 1024,
        ),
    )(exch, own)
